# baseline (speedup 1.0000x reference)
import functools

import jax
import jax.numpy as jnp
from jax import lax
from jax.experimental import pallas as pl
from jax.experimental.pallas import tpu as pltpu
from jax.experimental.pallas import tpu_sc as plsc

N = 10000
PR = N // 2
E_UP = 160000
E_B = 20000
D = 256
H = 256
NC = 2
NS = 16
QW = 64
CH = 64
PROW_BLK = 320
PROW_LAST = PR - 15 * PROW_BLK
EPT = E_UP // NS
UP_CHUNKS_PER_TILE = EPT // CH
UP_REM = EPT - UP_CHUNKS_PER_TILE * CH
B_CHUNKS = E_B // CH
B_REM = E_B - B_CHUNKS * CH


def _p1_body(xlo_ref, xhi_ref, blo_ref, bhi_ref, w_ref, b_ref,
             xa_out, xh_out, bh_out):
    dot = functools.partial(jnp.dot, preferred_element_type=jnp.float32)
    alo = dot(xlo_ref[...], w_ref[...]) + b_ref[...]
    ahi = dot(xhi_ref[...], w_ref[...]) + b_ref[...]
    for q in range(4):
        sl = slice(q * QW, (q + 1) * QW)
        xa_out[q] = jnp.concatenate([alo[:, sl], ahi[:, sl]], axis=1)
        xh_out[q] = jnp.concatenate([xlo_ref[:, sl], xhi_ref[:, sl]], axis=1)
        bh_out[q] = jnp.concatenate([blo_ref[:, sl], bhi_ref[:, sl]], axis=1)


def _run_p1(x, boundary_attr, W_mu, b_mu):
    blk = 1000
    nb = PR // blk
    pair_spec = pl.BlockSpec((4, blk, 2 * QW), lambda i: (0, i, 0))
    out = pl.pallas_call(
        _p1_body,
        grid=(nb,),
        in_specs=[
            pl.BlockSpec((blk, D), lambda i: (i, 0)),
            pl.BlockSpec((blk, D), lambda i: (i + nb, 0)),
            pl.BlockSpec((blk, D), lambda i: (i, 0)),
            pl.BlockSpec((blk, D), lambda i: (i + nb, 0)),
            pl.BlockSpec((D, D), lambda i: (0, 0)),
            pl.BlockSpec((1, D), lambda i: (0, 0)),
        ],
        out_specs=[pair_spec, pair_spec, pair_spec],
        out_shape=[jax.ShapeDtypeStruct((4, PR, 2 * QW), jnp.float32)] * 3,
    )(x, x, boundary_attr, boundary_attr, W_mu[:D], b_mu.reshape(1, D))
    xa, xh, bh = out
    return (xa.reshape(4 * PR, 2 * QW), xh.reshape(4 * PR, 2 * QW),
            bh.reshape(4 * PR, 2 * QW))


def _p2_body(ua_ref, w_ref, t_out):
    r = jnp.dot(ua_ref[...], w_ref[...], preferred_element_type=jnp.float32)
    for q in range(4):
        t_out[q] = r[:, q * QW:(q + 1) * QW]


def _run_p2(up_attr, W_mu):
    blk = 2000
    t = pl.pallas_call(
        _p2_body,
        grid=(E_UP // blk,),
        in_specs=[
            pl.BlockSpec((blk, D), lambda i: (i, 0)),
            pl.BlockSpec((D, D), lambda i: (1, 0)),
        ],
        out_specs=pl.BlockSpec((4, blk, QW), lambda i: (0, i, 0)),
        out_shape=jax.ShapeDtypeStruct((4, E_UP, QW), jnp.float32),
    )(up_attr, W_mu)
    return t.reshape(4 * E_UP, QW)


def _bcast_lane(vec, j):
    idx = jnp.full((16,), j, jnp.int32)
    return lax.gather(
        vec, idx[:, None],
        lax.GatherDimensionNumbers(offset_dims=(), collapsed_slice_dims=(0,),
                                   start_index_map=(0,)),
        (1,), mode=lax.GatherScatterMode.PROMISE_IN_BOUNDS)


def _sc_body(xa, t, xh, ba, usrc, udst, bsrc, bdst, ou, ob,
             acc, tab, t0, t1, g0, g1, sv0, sv1, dv0, dv1,
             sr0, sr1, dr0, dr1, ge, sve, dve, sre, dre,
             semL0, semL1, semG0, semG1, semB):
    c = lax.axis_index("c")
    s = lax.axis_index("s")
    r0 = s * PROW_BLK
    ebase = s * EPT

    tb = (t0, t1)
    gb = (g0, g1)
    sv = (sv0, sv1)
    dv = (dv0, dv1)
    sr = (sr0, sr1)
    dr = (dr0, dr1)
    semL = (semL0, semL1)
    semG = (semG0, semG1)

    prsplat = jnp.full((16,), PR, jnp.int32)

    def rows_copy(src_ref, dst_ref, soff, doff):
        @pl.when(s < NS - 1)
        def _():
            pltpu.sync_copy(src_ref.at[pl.ds(soff, PROW_BLK)],
                            dst_ref.at[pl.ds(doff, PROW_BLK)])

        @pl.when(s == NS - 1)
        def _():
            pltpu.sync_copy(src_ref.at[pl.ds(soff, PROW_LAST)],
                            dst_ref.at[pl.ds(doff, PROW_LAST)])

    def issue_loads(b, j, tqoff):
        base = ebase + j * CH
        pltpu.async_copy(t.at[pl.ds(tqoff + base, CH)], tb[b], semL[b])
        pltpu.async_copy(usrc.at[pl.ds(base, CH)], sv[b], semL[b])
        pltpu.async_copy(udst.at[pl.ds(base, CH)], dv[b], semL[b])

    def wait_loads(b, tqoff):
        pltpu.make_async_copy(t.at[pl.ds(tqoff + ebase, CH)], tb[b],
                              semL[b]).wait()
        pltpu.make_async_copy(usrc.at[pl.ds(ebase, CH)], sv[b],
                              semL[b]).wait()
        pltpu.make_async_copy(udst.at[pl.ds(ebase, CH)], dv[b],
                              semL[b]).wait()

    def rows_from_raw(b):
        _sv, _dv, _sr, _dr = sv[b], dv[b], sr[b], dr[b]

        def go(g, _):
            sl = pl.ds(g * 16, 16)
            _sr[sl] = lax.rem(_sv[sl], prsplat)
            _dr[sl] = lax.rem(_dv[sl], prsplat)
            return 0

        lax.fori_loop(0, CH // 16, go, 0)

    def issue_gather(b):
        pltpu.async_copy(tab.at[sr[b]], gb[b], semG[b])

    def wait_gather(b):
        pltpu.make_async_copy(tab.at[sr[b]], gb[b], semG[b]).wait()

    def blend16(buf, tbuf, psv, pdv, e0, with_t):
        def inner(j, _):
            e = e0 + j
            sm = _bcast_lane(psv, j)
            dm = _bcast_lane(pdv, j)
            dn = 1.0 - dm
            for kk in range(4):
                lo = pl.ds(kk * 16, 16)
                hi = pl.ds(QW + kk * 16, 16)
                glo = buf[e, lo]
                ghi = buf[e, hi]
                v = glo + sm * (ghi - glo)
                if with_t:
                    v = jnp.maximum(v + tbuf[e, lo], 0.0)
                buf[e, lo] = v * dn
                buf[e, hi] = v * dm
            return 0

        lax.fori_loop(0, 16, inner, 0)

    def compute_scatter(b, with_t):
        _tb, _gb, _sv, _dv = tb[b], gb[b], sv[b], dv[b]

        def group(g, _):
            slg = pl.ds(g * 16, 16)
            psv = lax.div(_sv[slg], prsplat).astype(jnp.float32)
            pdv = lax.div(_dv[slg], prsplat).astype(jnp.float32)
            blend16(_gb, _tb, psv, pdv, g * 16, with_t)
            return 0

        lax.fori_loop(0, CH // 16, group, 0)
        pltpu.sync_copy(_gb, acc.at[dr[b]], add=True)

    def mini16(sref, dref, base, tqoff, with_t):
        if with_t:
            pltpu.sync_copy(t.at[pl.ds(tqoff + base, 16)],
                            t0.at[pl.ds(0, 16)])
        pltpu.sync_copy(sref.at[pl.ds(base, 16)], sve)
        pltpu.sync_copy(dref.at[pl.ds(base, 16)], dve)
        sre[pl.ds(0, 16)] = lax.rem(sve[pl.ds(0, 16)], prsplat)
        dre[pl.ds(0, 16)] = lax.rem(dve[pl.ds(0, 16)], prsplat)
        pltpu.async_copy(tab.at[sre], ge, semB).wait()
        psv = lax.div(sve[pl.ds(0, 16)], prsplat).astype(jnp.float32)
        pdv = lax.div(dve[pl.ds(0, 16)], prsplat).astype(jnp.float32)
        blend16(ge, t0, psv, pdv, 0, with_t)
        pltpu.sync_copy(ge, acc.at[dre], add=True)

    NPAIR = UP_CHUNKS_PER_TILE // 2

    def phase(qi, _):
        qoff = (2 * c + qi) * PR
        tqoff = (2 * c + qi) * E_UP

        rows_copy(xa, tab, qoff + r0, r0)
        rows_copy(xh, acc, qoff + r0, r0)
        plsc.subcore_barrier()

        issue_loads(0, 0, tqoff)
        wait_loads(0, tqoff)
        rows_from_raw(0)
        issue_gather(0)
        issue_loads(1, 1, tqoff)

        def pair(p, _):
            wait_gather(0)
            wait_loads(1, tqoff)
            rows_from_raw(1)
            issue_gather(1)
            compute_scatter(0, True)

            @pl.when(p < NPAIR - 1)
            def _():
                issue_loads(0, 2 * p + 2, tqoff)

            wait_gather(1)
            compute_scatter(1, True)

            @pl.when(p < NPAIR - 1)
            def _():
                wait_loads(0, tqoff)
                rows_from_raw(0)
                issue_gather(0)
                issue_loads(1, 2 * p + 3, tqoff)

            return 0

        lax.fori_loop(0, NPAIR, pair, 0)

        mini16(usrc, udst, ebase + UP_CHUNKS_PER_TILE * CH, tqoff, True)

        plsc.subcore_barrier()
        rows_copy(acc, ou, r0, qoff + r0)

        rows_copy(ba, tab, qoff + r0, r0)
        rows_copy(xh, acc, qoff + r0, r0)
        plsc.subcore_barrier()

        nk = (B_CHUNKS - s + NS - 1) // NS

        def b_chunk(jj, _):
            base = (s + jj * NS) * CH
            pltpu.sync_copy(bsrc.at[pl.ds(base, CH)], sv0)
            pltpu.sync_copy(bdst.at[pl.ds(base, CH)], dv0)
            rows_from_raw(0)
            pltpu.async_copy(tab.at[sr0], g0, semB).wait()
            compute_scatter(0, False)
            return 0

        lax.fori_loop(0, nk, b_chunk, 0)

        @pl.when(s == 0)
        def _():
            def go(r16, _):
                mini16(bsrc, bdst, B_CHUNKS * CH + r16 * 16, 0, False)
                return 0

            lax.fori_loop(0, 2, go, 0)

        plsc.subcore_barrier()
        rows_copy(acc, ob, r0, qoff + r0)
        plsc.subcore_barrier()
        return 0

    lax.fori_loop(0, 2, phase, 0)


def _run_sc(xa, t, xh, ba, up_index, boundary_index):
    mesh = plsc.VectorSubcoreMesh(core_axis_name="c", subcore_axis_name="s",
                                  num_cores=NC, num_subcores=NS)
    f = pl.kernel(
        _sc_body,
        out_type=[jax.ShapeDtypeStruct((4 * PR, 2 * QW), jnp.float32)] * 2,
        mesh=mesh,
        scratch_types=[
            pltpu.VMEM_SHARED((PR, 2 * QW), jnp.float32),
            pltpu.VMEM_SHARED((PR, 2 * QW), jnp.float32),
            pltpu.VMEM((CH, QW), jnp.float32),
            pltpu.VMEM((CH, QW), jnp.float32),
            pltpu.VMEM((CH, 2 * QW), jnp.float32),
            pltpu.VMEM((CH, 2 * QW), jnp.float32),
            pltpu.VMEM((CH,), jnp.int32),
            pltpu.VMEM((CH,), jnp.int32),
            pltpu.VMEM((CH,), jnp.int32),
            pltpu.VMEM((CH,), jnp.int32),
            pltpu.VMEM((CH,), jnp.int32),
            pltpu.VMEM((CH,), jnp.int32),
            pltpu.VMEM((CH,), jnp.int32),
            pltpu.VMEM((CH,), jnp.int32),
            pltpu.VMEM((16, 2 * QW), jnp.float32),
            pltpu.VMEM((16,), jnp.int32),
            pltpu.VMEM((16,), jnp.int32),
            pltpu.VMEM((16,), jnp.int32),
            pltpu.VMEM((16,), jnp.int32),
            pltpu.SemaphoreType.DMA,
            pltpu.SemaphoreType.DMA,
            pltpu.SemaphoreType.DMA,
            pltpu.SemaphoreType.DMA,
            pltpu.SemaphoreType.DMA,
        ],
    )
    return f(xa, t, xh, ba, up_index[1], up_index[0],
             boundary_index[0], boundary_index[1])


def _bn_relu(h, g, beta):
    mu = jnp.mean(h, axis=0, keepdims=True)
    d = h - mu
    var = jnp.mean(d * d, axis=0, keepdims=True)
    return jnp.maximum(g * d * lax.rsqrt(var + 1e-5) + beta, 0.0)


def _qdot(stacked_ref, w_ref):
    dot = functools.partial(jnp.dot, preferred_element_type=jnp.float32)
    lo = dot(stacked_ref[0:PR, :QW], w_ref[0:QW])
    hi = dot(stacked_ref[0:PR, QW:], w_ref[0:QW])
    for q in range(1, 4):
        rows = slice(q * PR, (q + 1) * PR)
        wrows = slice(q * QW, (q + 1) * QW)
        lo += dot(stacked_ref[rows, :QW], w_ref[wrows])
        hi += dot(stacked_ref[rows, QW:], w_ref[wrows])
    return jnp.concatenate([lo, hi], axis=0)


def _branch_body(in_ref, w1, b1, w2, b2, g1, beta1, g2, beta2, out_ref):
    dot = functools.partial(jnp.dot, preferred_element_type=jnp.float32)
    u = _qdot(in_ref, w1) + b1[...]
    u = _bn_relu(u, g1[...], beta1[...])
    out_ref[...] = _bn_relu(dot(u, w2[...]) + b2[...], g2[...], beta2[...])


def _run_branch(stacked, W1, b1, W2, b2, g1, beta1, g2, beta2):
    row = lambda v: v.reshape(1, H)
    return pl.pallas_call(
        _branch_body,
        out_shape=jax.ShapeDtypeStruct((N, H), jnp.float32),
    )(stacked, W1, row(b1), W2, row(b2), row(g1), row(beta1),
      row(g2), row(beta2))


def _combine_body(u_ref, b_ref, wc, bc, gc, betac, out_ref):
    dot = functools.partial(jnp.dot, preferred_element_type=jnp.float32)
    o = dot(u_ref[...], wc[:H]) + dot(b_ref[...], wc[H:]) + bc[...]
    out_ref[...] = _bn_relu(o, gc[...], betac[...])


def _run_combine(u2, b2, Wc, bc, gc, betac):
    row = lambda v: v.reshape(1, H)
    return pl.pallas_call(
        _combine_body,
        out_shape=jax.ShapeDtypeStruct((N, H), jnp.float32),
    )(u2, b2, Wc, row(bc), row(gc), row(betac))


def _run_mlp(ou, ob, Wu1, bu1, Wu2, bu2, Wb1, bb1, Wb2, bb2, Wc, bc,
             gu1, betau1, gu2, betau2, gb1, betab1, gb2, betab2, gc, betac):
    u2 = _run_branch(ou, Wu1, bu1, Wu2, bu2, gu1, betau1, gu2, betau2)
    b2 = _run_branch(ob, Wb1, bb1, Wb2, bb2, gb1, betab1, gb2, betab2)
    return _run_combine(u2, b2, Wc, bc, gc, betac)


def kernel(x, up_index, up_attr, boundary_index, boundary_attr, W_mu, b_mu,
           Wu1, bu1, Wu2, bu2, Wb1, bb1, Wb2, bb2, Wc, bc, gu1, betau1,
           gu2, betau2, gb1, betab1, gb2, betab2, gc, betac):
    xa, xh, bh = _run_p1(x, boundary_attr, W_mu, b_mu)
    t = _run_p2(up_attr, W_mu)
    ou, ob = _run_sc(xa, t, xh, bh, up_index, boundary_index)
    return _run_mlp(ou, ob, Wu1, bu1, Wu2, bu2, Wb1, bb1, Wb2, bb2, Wc, bc,
                    gu1, betau1, gu2, betau2, gb1, betab1, gb2, betab2,
                    gc, betac)


# speedup vs baseline: 1.2780x; 1.2780x over previous
import functools

import jax
import jax.numpy as jnp
from jax import lax
from jax.experimental import pallas as pl
from jax.experimental.pallas import tpu as pltpu
from jax.experimental.pallas import tpu_sc as plsc

N = 10000
PR = N // 2
E_UP = 160000
E_B = 20000
D = 256
H = 256
NC = 2
NS = 16
QW = 64
CH = 64
PROW_BLK = 320
PROW_LAST = PR - 15 * PROW_BLK
EPT = E_UP // NS
UP_CHUNKS_PER_TILE = EPT // CH
UP_REM = EPT - UP_CHUNKS_PER_TILE * CH
B_CHUNKS = E_B // CH
B_REM = E_B - B_CHUNKS * CH


def _p1_body(xlo_ref, xhi_ref, blo_ref, bhi_ref, w_ref, b_ref,
             xa_out, xh_out, bh_out):
    dot = functools.partial(jnp.dot, preferred_element_type=jnp.float32)
    alo = dot(xlo_ref[...], w_ref[...]) + b_ref[...]
    ahi = dot(xhi_ref[...], w_ref[...]) + b_ref[...]
    for q in range(4):
        sl = slice(q * QW, (q + 1) * QW)
        xa_out[q] = jnp.concatenate([alo[:, sl], ahi[:, sl]], axis=1)
        xh_out[q] = jnp.concatenate([xlo_ref[:, sl], xhi_ref[:, sl]], axis=1)
        bh_out[q] = jnp.concatenate([blo_ref[:, sl], bhi_ref[:, sl]], axis=1)


def _run_p1(x, boundary_attr, W_mu, b_mu):
    blk = 1000
    nb = PR // blk
    pair_spec = pl.BlockSpec((4, blk, 2 * QW), lambda i: (0, i, 0))
    out = pl.pallas_call(
        _p1_body,
        grid=(nb,),
        in_specs=[
            pl.BlockSpec((blk, D), lambda i: (i, 0)),
            pl.BlockSpec((blk, D), lambda i: (i + nb, 0)),
            pl.BlockSpec((blk, D), lambda i: (i, 0)),
            pl.BlockSpec((blk, D), lambda i: (i + nb, 0)),
            pl.BlockSpec((D, D), lambda i: (0, 0)),
            pl.BlockSpec((1, D), lambda i: (0, 0)),
        ],
        out_specs=[pair_spec, pair_spec, pair_spec],
        out_shape=[jax.ShapeDtypeStruct((4, PR, 2 * QW), jnp.float32)] * 3,
    )(x, x, boundary_attr, boundary_attr, W_mu[:D], b_mu.reshape(1, D))
    xa, xh, bh = out
    return (xa.reshape(4 * PR, 2 * QW), xh.reshape(4 * PR, 2 * QW),
            bh.reshape(4 * PR, 2 * QW))


def _p2_body(ua_ref, w_ref, t_out):
    r = jnp.dot(ua_ref[...], w_ref[...], preferred_element_type=jnp.float32)
    for q in range(4):
        t_out[q] = r[:, q * QW:(q + 1) * QW]


def _run_p2(up_attr, W_mu):
    blk = 2000
    t = pl.pallas_call(
        _p2_body,
        grid=(E_UP // blk,),
        in_specs=[
            pl.BlockSpec((blk, D), lambda i: (i, 0)),
            pl.BlockSpec((D, D), lambda i: (1, 0)),
        ],
        out_specs=pl.BlockSpec((4, blk, QW), lambda i: (0, i, 0)),
        out_shape=jax.ShapeDtypeStruct((4, E_UP, QW), jnp.float32),
    )(up_attr, W_mu)
    return t.reshape(4 * E_UP, QW)


def _bcast_lane(vec, j):
    idx = jnp.full((16,), j, jnp.int32)
    return lax.gather(
        vec, idx[:, None],
        lax.GatherDimensionNumbers(offset_dims=(), collapsed_slice_dims=(0,),
                                   start_index_map=(0,)),
        (1,), mode=lax.GatherScatterMode.PROMISE_IN_BOUNDS)


def _sc_body(xa, t, xh, ba, usrc, udst, bsrc, bdst, ou, ob,
             acc, tab, t0, t1, g0, g1, sv0, sv1, dv0, dv1,
             sr0, sr1, dr0, dr1, ge, sve, dve, sre, dre,
             semL0, semL1, semG0, semG1, semB):
    c = lax.axis_index("c")
    s = lax.axis_index("s")
    r0 = s * PROW_BLK
    ebase = s * EPT

    tb = (t0, t1)
    gb = (g0, g1)
    sv = (sv0, sv1)
    dv = (dv0, dv1)
    sr = (sr0, sr1)
    dr = (dr0, dr1)
    semL = (semL0, semL1)
    semG = (semG0, semG1)

    prsplat = jnp.full((16,), PR, jnp.int32)

    def rows_copy(src_ref, dst_ref, soff, doff):
        @pl.when(s < NS - 1)
        def _():
            pltpu.sync_copy(src_ref.at[pl.ds(soff, PROW_BLK)],
                            dst_ref.at[pl.ds(doff, PROW_BLK)])

        @pl.when(s == NS - 1)
        def _():
            pltpu.sync_copy(src_ref.at[pl.ds(soff, PROW_LAST)],
                            dst_ref.at[pl.ds(doff, PROW_LAST)])

    def issue_loads(b, j, tqoff):
        base = ebase + j * CH
        pltpu.async_copy(t.at[pl.ds(tqoff + base, CH)], tb[b], semL[b])
        pltpu.async_copy(usrc.at[pl.ds(base, CH)], sv[b], semL[b])
        pltpu.async_copy(udst.at[pl.ds(base, CH)], dv[b], semL[b])

    def wait_loads(b, tqoff):
        pltpu.make_async_copy(t.at[pl.ds(tqoff + ebase, CH)], tb[b],
                              semL[b]).wait()
        pltpu.make_async_copy(usrc.at[pl.ds(ebase, CH)], sv[b],
                              semL[b]).wait()
        pltpu.make_async_copy(udst.at[pl.ds(ebase, CH)], dv[b],
                              semL[b]).wait()

    def rows_from_raw(b):
        _sv, _dv, _sr, _dr = sv[b], dv[b], sr[b], dr[b]

        def go(g, _):
            sl = pl.ds(g * 16, 16)
            _sr[sl] = lax.rem(_sv[sl], prsplat)
            _dr[sl] = lax.rem(_dv[sl], prsplat)
            return 0

        lax.fori_loop(0, CH // 16, go, 0)

    def issue_gather(b):
        pltpu.async_copy(tab.at[sr[b]], gb[b], semG[b])

    def wait_gather(b):
        pltpu.make_async_copy(tab.at[sr[b]], gb[b], semG[b]).wait()

    def blend16(buf, tbuf, psv, pdv, e0, with_t):
        for j in range(16):
            e = e0 + j
            sm = _bcast_lane(psv, j)
            dm = _bcast_lane(pdv, j)
            dn = 1.0 - dm
            for kk in range(4):
                lo = pl.ds(kk * 16, 16)
                hi = pl.ds(QW + kk * 16, 16)
                glo = buf[e, lo]
                ghi = buf[e, hi]
                v = glo + sm * (ghi - glo)
                if with_t:
                    v = jnp.maximum(v + tbuf[e, lo], 0.0)
                buf[e, lo] = v * dn
                buf[e, hi] = v * dm

    def compute_scatter(b, with_t):
        _tb, _gb, _sv, _dv = tb[b], gb[b], sv[b], dv[b]

        @plsc.parallel_loop(0, CH // 16, unroll=1)
        def _(g):
            slg = pl.ds(g * 16, 16)
            psv = lax.div(_sv[slg], prsplat).astype(jnp.float32)
            pdv = lax.div(_dv[slg], prsplat).astype(jnp.float32)
            blend16(_gb, _tb, psv, pdv, g * 16, with_t)

        pltpu.sync_copy(_gb, acc.at[dr[b]], add=True)

    def mini16(sref, dref, base, tqoff, with_t):
        if with_t:
            pltpu.sync_copy(t.at[pl.ds(tqoff + base, 16)],
                            t0.at[pl.ds(0, 16)])
        pltpu.sync_copy(sref.at[pl.ds(base, 16)], sve)
        pltpu.sync_copy(dref.at[pl.ds(base, 16)], dve)
        sre[pl.ds(0, 16)] = lax.rem(sve[pl.ds(0, 16)], prsplat)
        dre[pl.ds(0, 16)] = lax.rem(dve[pl.ds(0, 16)], prsplat)
        pltpu.async_copy(tab.at[sre], ge, semB).wait()
        psv = lax.div(sve[pl.ds(0, 16)], prsplat).astype(jnp.float32)
        pdv = lax.div(dve[pl.ds(0, 16)], prsplat).astype(jnp.float32)
        blend16(ge, t0, psv, pdv, 0, with_t)
        pltpu.sync_copy(ge, acc.at[dre], add=True)

    NPAIR = UP_CHUNKS_PER_TILE // 2

    def phase(qi, _):
        qoff = (2 * c + qi) * PR
        tqoff = (2 * c + qi) * E_UP

        rows_copy(xa, tab, qoff + r0, r0)
        rows_copy(xh, acc, qoff + r0, r0)
        plsc.subcore_barrier()

        issue_loads(0, 0, tqoff)
        wait_loads(0, tqoff)
        rows_from_raw(0)
        issue_gather(0)
        issue_loads(1, 1, tqoff)

        def pair(p, _):
            wait_gather(0)
            wait_loads(1, tqoff)
            rows_from_raw(1)
            issue_gather(1)
            compute_scatter(0, True)

            @pl.when(p < NPAIR - 1)
            def _():
                issue_loads(0, 2 * p + 2, tqoff)

            wait_gather(1)
            compute_scatter(1, True)

            @pl.when(p < NPAIR - 1)
            def _():
                wait_loads(0, tqoff)
                rows_from_raw(0)
                issue_gather(0)
                issue_loads(1, 2 * p + 3, tqoff)

            return 0

        lax.fori_loop(0, NPAIR, pair, 0)

        mini16(usrc, udst, ebase + UP_CHUNKS_PER_TILE * CH, tqoff, True)

        plsc.subcore_barrier()
        rows_copy(acc, ou, r0, qoff + r0)

        rows_copy(ba, tab, qoff + r0, r0)
        rows_copy(xh, acc, qoff + r0, r0)
        plsc.subcore_barrier()

        nk = (B_CHUNKS - s + NS - 1) // NS

        def b_chunk(jj, _):
            base = (s + jj * NS) * CH
            pltpu.sync_copy(bsrc.at[pl.ds(base, CH)], sv0)
            pltpu.sync_copy(bdst.at[pl.ds(base, CH)], dv0)
            rows_from_raw(0)
            pltpu.async_copy(tab.at[sr0], g0, semB).wait()
            compute_scatter(0, False)
            return 0

        lax.fori_loop(0, nk, b_chunk, 0)

        @pl.when(s == 0)
        def _():
            def go(r16, _):
                mini16(bsrc, bdst, B_CHUNKS * CH + r16 * 16, 0, False)
                return 0

            lax.fori_loop(0, 2, go, 0)

        plsc.subcore_barrier()
        rows_copy(acc, ob, r0, qoff + r0)
        plsc.subcore_barrier()
        return 0

    lax.fori_loop(0, 2, phase, 0)


def _run_sc(xa, t, xh, ba, up_index, boundary_index):
    mesh = plsc.VectorSubcoreMesh(core_axis_name="c", subcore_axis_name="s",
                                  num_cores=NC, num_subcores=NS)
    f = pl.kernel(
        _sc_body,
        out_type=[jax.ShapeDtypeStruct((4 * PR, 2 * QW), jnp.float32)] * 2,
        mesh=mesh,
        scratch_types=[
            pltpu.VMEM_SHARED((PR, 2 * QW), jnp.float32),
            pltpu.VMEM_SHARED((PR, 2 * QW), jnp.float32),
            pltpu.VMEM((CH, QW), jnp.float32),
            pltpu.VMEM((CH, QW), jnp.float32),
            pltpu.VMEM((CH, 2 * QW), jnp.float32),
            pltpu.VMEM((CH, 2 * QW), jnp.float32),
            pltpu.VMEM((CH,), jnp.int32),
            pltpu.VMEM((CH,), jnp.int32),
            pltpu.VMEM((CH,), jnp.int32),
            pltpu.VMEM((CH,), jnp.int32),
            pltpu.VMEM((CH,), jnp.int32),
            pltpu.VMEM((CH,), jnp.int32),
            pltpu.VMEM((CH,), jnp.int32),
            pltpu.VMEM((CH,), jnp.int32),
            pltpu.VMEM((16, 2 * QW), jnp.float32),
            pltpu.VMEM((16,), jnp.int32),
            pltpu.VMEM((16,), jnp.int32),
            pltpu.VMEM((16,), jnp.int32),
            pltpu.VMEM((16,), jnp.int32),
            pltpu.SemaphoreType.DMA,
            pltpu.SemaphoreType.DMA,
            pltpu.SemaphoreType.DMA,
            pltpu.SemaphoreType.DMA,
            pltpu.SemaphoreType.DMA,
        ],
    )
    return f(xa, t, xh, ba, up_index[1], up_index[0],
             boundary_index[0], boundary_index[1])


def _bn_relu(h, g, beta):
    mu = jnp.mean(h, axis=0, keepdims=True)
    d = h - mu
    var = jnp.mean(d * d, axis=0, keepdims=True)
    return jnp.maximum(g * d * lax.rsqrt(var + 1e-5) + beta, 0.0)


def _qdot(stacked_ref, w_ref):
    dot = functools.partial(jnp.dot, preferred_element_type=jnp.float32)
    lo = dot(stacked_ref[0:PR, :QW], w_ref[0:QW])
    hi = dot(stacked_ref[0:PR, QW:], w_ref[0:QW])
    for q in range(1, 4):
        rows = slice(q * PR, (q + 1) * PR)
        wrows = slice(q * QW, (q + 1) * QW)
        lo += dot(stacked_ref[rows, :QW], w_ref[wrows])
        hi += dot(stacked_ref[rows, QW:], w_ref[wrows])
    return jnp.concatenate([lo, hi], axis=0)


def _branch_body(in_ref, w1, b1, w2, b2, g1, beta1, g2, beta2, out_ref):
    dot = functools.partial(jnp.dot, preferred_element_type=jnp.float32)
    u = _qdot(in_ref, w1) + b1[...]
    u = _bn_relu(u, g1[...], beta1[...])
    out_ref[...] = _bn_relu(dot(u, w2[...]) + b2[...], g2[...], beta2[...])


def _run_branch(stacked, W1, b1, W2, b2, g1, beta1, g2, beta2):
    row = lambda v: v.reshape(1, H)
    return pl.pallas_call(
        _branch_body,
        out_shape=jax.ShapeDtypeStruct((N, H), jnp.float32),
    )(stacked, W1, row(b1), W2, row(b2), row(g1), row(beta1),
      row(g2), row(beta2))


def _combine_body(u_ref, b_ref, wc, bc, gc, betac, out_ref):
    dot = functools.partial(jnp.dot, preferred_element_type=jnp.float32)
    o = dot(u_ref[...], wc[:H]) + dot(b_ref[...], wc[H:]) + bc[...]
    out_ref[...] = _bn_relu(o, gc[...], betac[...])


def _run_combine(u2, b2, Wc, bc, gc, betac):
    row = lambda v: v.reshape(1, H)
    return pl.pallas_call(
        _combine_body,
        out_shape=jax.ShapeDtypeStruct((N, H), jnp.float32),
    )(u2, b2, Wc, row(bc), row(gc), row(betac))


def _run_mlp(ou, ob, Wu1, bu1, Wu2, bu2, Wb1, bb1, Wb2, bb2, Wc, bc,
             gu1, betau1, gu2, betau2, gb1, betab1, gb2, betab2, gc, betac):
    u2 = _run_branch(ou, Wu1, bu1, Wu2, bu2, gu1, betau1, gu2, betau2)
    b2 = _run_branch(ob, Wb1, bb1, Wb2, bb2, gb1, betab1, gb2, betab2)
    return _run_combine(u2, b2, Wc, bc, gc, betac)


def kernel(x, up_index, up_attr, boundary_index, boundary_attr, W_mu, b_mu,
           Wu1, bu1, Wu2, bu2, Wb1, bb1, Wb2, bb2, Wc, bc, gu1, betau1,
           gu2, betau2, gb1, betab1, gb2, betab2, gc, betac):
    xa, xh, bh = _run_p1(x, boundary_attr, W_mu, b_mu)
    t = _run_p2(up_attr, W_mu)
    ou, ob = _run_sc(xa, t, xh, bh, up_index, boundary_index)
    return _run_mlp(ou, ob, Wu1, bu1, Wu2, bu2, Wb1, bb1, Wb2, bb2, Wc, bc,
                    gu1, betau1, gu2, betau2, gb1, betab1, gb2, betab2,
                    gc, betac)


# speedup vs baseline: 2.7161x; 2.1252x over previous
import functools

import jax
import jax.numpy as jnp
from jax import lax
from jax.experimental import pallas as pl
from jax.experimental.pallas import tpu as pltpu
from jax.experimental.pallas import tpu_sc as plsc

N = 10000
E_UP = 160000
E_B = 20000
D = 256
H = 256
NC = 2
NS = 16
CH = 64
ROW_BLK = 640
ROW_LAST = N - 15 * ROW_BLK
EPT = E_UP // NS
UP_CHUNKS_PER_TILE = EPT // CH
UP_REM = EPT - UP_CHUNKS_PER_TILE * CH
CHB = 80
B_CHUNKS = E_B // CHB


def _p1_body(x_ref, ba_ref, w_ref, b_ref, xa_out, xh_out, bh_out):
    a = jnp.dot(x_ref[...], w_ref[...], preferred_element_type=jnp.float32)
    a = a + b_ref[...]
    xa_out[0] = a[:, :128]
    xa_out[1] = a[:, 128:]
    xh_out[0] = x_ref[:, :128]
    xh_out[1] = x_ref[:, 128:]
    bh_out[0] = ba_ref[:, :128]
    bh_out[1] = ba_ref[:, 128:]


def _run_p1(x, boundary_attr, W_mu, b_mu):
    blk = 1000
    grid = (N // blk,)
    out = pl.pallas_call(
        _p1_body,
        grid=grid,
        in_specs=[
            pl.BlockSpec((blk, D), lambda i: (i, 0)),
            pl.BlockSpec((blk, D), lambda i: (i, 0)),
            pl.BlockSpec((D, D), lambda i: (0, 0)),
            pl.BlockSpec((1, D), lambda i: (0, 0)),
        ],
        out_specs=[
            pl.BlockSpec((2, blk, 128), lambda i: (0, i, 0)),
            pl.BlockSpec((2, blk, 128), lambda i: (0, i, 0)),
            pl.BlockSpec((2, blk, 128), lambda i: (0, i, 0)),
        ],
        out_shape=[jax.ShapeDtypeStruct((2, N, 128), jnp.float32)] * 3,
    )(x, boundary_attr, W_mu[:D], b_mu.reshape(1, D))
    xa, xh, bh = out
    return (xa.reshape(2 * N, 128), xh.reshape(2 * N, 128),
            bh.reshape(2 * N, 128))


def _p2_body(ua_ref, w_ref, t_out):
    r = jnp.dot(ua_ref[...], w_ref[...], preferred_element_type=jnp.float32)
    t_out[0] = r[:, :128]
    t_out[1] = r[:, 128:]


def _run_p2(up_attr, W_mu):
    blk = 2000
    grid = (E_UP // blk,)
    t = pl.pallas_call(
        _p2_body,
        grid=grid,
        in_specs=[
            pl.BlockSpec((blk, D), lambda i: (i, 0)),
            pl.BlockSpec((D, D), lambda i: (1, 0)),
        ],
        out_specs=pl.BlockSpec((2, blk, 128), lambda i: (0, i, 0)),
        out_shape=jax.ShapeDtypeStruct((2, E_UP, 128), jnp.float32),
    )(up_attr, W_mu)
    return t.reshape(2 * E_UP, 128)


def _sc_body(xa, t, xh, ba, usrc2, udst, bsrc2, bdst, ou, ob,
             acc, t0, t1, g0, g1, s0, s1, d0, d1,
             te, ge, se, de, gb_b, sb_b, db_b,
             semL0, semL1, semG0, semG1, semB):
    c = lax.axis_index("c")
    s = lax.axis_index("s")
    r0 = s * ROW_BLK
    xoff = c * N
    toff = c * E_UP
    ebase = s * EPT

    tb = (t0, t1)
    gb = (g0, g1)
    sb = (s0, s1)
    db = (d0, d1)
    semL = (semL0, semL1)
    semG = (semG0, semG1)

    def rows_copy(src_ref, dst_ref, soff, doff):
        @pl.when(s < NS - 1)
        def _():
            pltpu.sync_copy(src_ref.at[pl.ds(soff, ROW_BLK)],
                            dst_ref.at[pl.ds(doff, ROW_BLK)])

        @pl.when(s == NS - 1)
        def _():
            pltpu.sync_copy(src_ref.at[pl.ds(soff, ROW_LAST)],
                            dst_ref.at[pl.ds(doff, ROW_LAST)])

    rows_copy(xh, acc, xoff + r0, r0)
    plsc.subcore_barrier()

    def issue_loads(b, j):
        base = ebase + j * CH
        pltpu.async_copy(t.at[pl.ds(toff + base, CH)], tb[b], semL[b])
        pltpu.async_copy(usrc2.at[pl.ds(toff + base, CH)], sb[b], semL[b])
        pltpu.async_copy(udst.at[pl.ds(base, CH)], db[b], semL[b])

    def wait_loads(b):
        pltpu.make_async_copy(t.at[pl.ds(toff + ebase, CH)], tb[b],
                              semL[b]).wait()
        pltpu.make_async_copy(usrc2.at[pl.ds(toff + ebase, CH)], sb[b],
                              semL[b]).wait()
        pltpu.make_async_copy(udst.at[pl.ds(ebase, CH)], db[b],
                              semL[b]).wait()

    def issue_gather(b):
        pltpu.async_copy(xa.at[sb[b]], gb[b], semG[b])

    def wait_gather(b):
        pltpu.make_async_copy(xa.at[sb[b]], gb[b], semG[b]).wait()

    def compute_scatter(b):
        _tb, _gb = tb[b], gb[b]

        @plsc.parallel_loop(0, CH, unroll=4)
        def _(r):
            for k in range(8):
                sl = pl.ds(k * 16, 16)
                _tb[r, sl] = jnp.maximum(_tb[r, sl] + _gb[r, sl], 0.0)

        pltpu.sync_copy(_tb, acc.at[db[b]], add=True)

    issue_loads(0, 0)
    wait_loads(0)
    issue_gather(0)
    issue_loads(1, 1)

    NP = UP_CHUNKS_PER_TILE // 2

    def pair(p, _):
        wait_gather(0)
        wait_loads(1)
        issue_gather(1)
        compute_scatter(0)

        @pl.when(p < NP - 1)
        def _():
            issue_loads(0, 2 * p + 2)

        wait_gather(1)
        compute_scatter(1)

        @pl.when(p < NP - 1)
        def _():
            wait_loads(0)
            issue_gather(0)
            issue_loads(1, 2 * p + 3)

        return 0

    lax.fori_loop(0, NP, pair, 0)

    rbase = ebase + UP_CHUNKS_PER_TILE * CH
    pltpu.sync_copy(t.at[pl.ds(toff + rbase, UP_REM)], te)
    pltpu.sync_copy(usrc2.at[pl.ds(toff + rbase, UP_REM)], se)
    pltpu.sync_copy(udst.at[pl.ds(rbase, UP_REM)], de)
    pltpu.async_copy(xa.at[se], ge, semB).wait()

    @plsc.parallel_loop(0, UP_REM, unroll=4)
    def _(r):
        for k in range(8):
            sl = pl.ds(k * 16, 16)
            te[r, sl] = jnp.maximum(te[r, sl] + ge[r, sl], 0.0)

    pltpu.sync_copy(te, acc.at[de], add=True)

    plsc.subcore_barrier()
    rows_copy(acc, ou, r0, xoff + r0)

    rows_copy(xh, acc, xoff + r0, r0)
    plsc.subcore_barrier()

    nk = (B_CHUNKS - s + NS - 1) // NS

    def b_chunk(j, _):
        base = (s + j * NS) * CHB
        pltpu.sync_copy(bsrc2.at[pl.ds(c * E_B + base, CHB)], sb_b)
        pltpu.sync_copy(bdst.at[pl.ds(base, CHB)], db_b)
        pltpu.async_copy(ba.at[sb_b], gb_b, semB).wait()
        pltpu.sync_copy(gb_b, acc.at[db_b], add=True)
        return 0

    lax.fori_loop(0, nk, b_chunk, 0)
    plsc.subcore_barrier()
    rows_copy(acc, ob, r0, xoff + r0)


def _run_sc(xa, t, xh, ba, up_index, boundary_index):
    mesh = plsc.VectorSubcoreMesh(core_axis_name="c", subcore_axis_name="s",
                                  num_cores=NC, num_subcores=NS)
    f = pl.kernel(
        _sc_body,
        out_type=[jax.ShapeDtypeStruct((2 * N, 128), jnp.float32)] * 2,
        mesh=mesh,
        scratch_types=[
            pltpu.VMEM_SHARED((N, 128), jnp.float32),
            pltpu.VMEM((CH, 128), jnp.float32),
            pltpu.VMEM((CH, 128), jnp.float32),
            pltpu.VMEM((CH, 128), jnp.float32),
            pltpu.VMEM((CH, 128), jnp.float32),
            pltpu.VMEM((CH,), jnp.int32),
            pltpu.VMEM((CH,), jnp.int32),
            pltpu.VMEM((CH,), jnp.int32),
            pltpu.VMEM((CH,), jnp.int32),
            pltpu.VMEM((UP_REM, 128), jnp.float32),
            pltpu.VMEM((UP_REM, 128), jnp.float32),
            pltpu.VMEM((UP_REM,), jnp.int32),
            pltpu.VMEM((UP_REM,), jnp.int32),
            pltpu.VMEM((CHB, 128), jnp.float32),
            pltpu.VMEM((CHB,), jnp.int32),
            pltpu.VMEM((CHB,), jnp.int32),
            pltpu.SemaphoreType.DMA,
            pltpu.SemaphoreType.DMA,
            pltpu.SemaphoreType.DMA,
            pltpu.SemaphoreType.DMA,
            pltpu.SemaphoreType.DMA,
        ],
    )
    usrc2 = jnp.concatenate([up_index[1], up_index[1] + N])
    bsrc2 = jnp.concatenate([boundary_index[0], boundary_index[0] + N])
    return f(xa, t, xh, ba, usrc2, up_index[0], bsrc2, boundary_index[1])


def _bn_relu(h, g, beta):
    mu = jnp.mean(h, axis=0, keepdims=True)
    d = h - mu
    var = jnp.mean(d * d, axis=0, keepdims=True)
    return jnp.maximum(g * d * lax.rsqrt(var + 1e-5) + beta, 0.0)


def _mlp_body(ou_ref, ob_ref, wu1, bu1, wu2, bu2, wb1, bb1, wb2, bb2,
              wc, bc, gu1, betau1, gu2, betau2, gb1, betab1, gb2, betab2,
              gc, betac, out_ref):
    dot = functools.partial(jnp.dot, preferred_element_type=jnp.float32)
    u = dot(ou_ref[:N], wu1[:128]) + dot(ou_ref[N:], wu1[128:]) + bu1[...]
    u = _bn_relu(u, gu1[...], betau1[...])
    u = _bn_relu(dot(u, wu2[...]) + bu2[...], gu2[...], betau2[...])
    b = dot(ob_ref[:N], wb1[:128]) + dot(ob_ref[N:], wb1[128:]) + bb1[...]
    b = _bn_relu(b, gb1[...], betab1[...])
    b = _bn_relu(dot(b, wb2[...]) + bb2[...], gb2[...], betab2[...])
    o = dot(u, wc[:H]) + dot(b, wc[H:]) + bc[...]
    out_ref[...] = _bn_relu(o, gc[...], betac[...])


def _run_mlp(ou, ob, Wu1, bu1, Wu2, bu2, Wb1, bb1, Wb2, bb2, Wc, bc,
             gu1, betau1, gu2, betau2, gb1, betab1, gb2, betab2, gc, betac):
    row = lambda v: v.reshape(1, H)
    return pl.pallas_call(
        _mlp_body,
        out_shape=jax.ShapeDtypeStruct((N, H), jnp.float32),
    )(ou, ob, Wu1, row(bu1), Wu2, row(bu2), Wb1, row(bb1), Wb2, row(bb2),
      Wc, row(bc), row(gu1), row(betau1), row(gu2), row(betau2),
      row(gb1), row(betab1), row(gb2), row(betab2), row(gc), row(betac))


def kernel(x, up_index, up_attr, boundary_index, boundary_attr, W_mu, b_mu,
           Wu1, bu1, Wu2, bu2, Wb1, bb1, Wb2, bb2, Wc, bc, gu1, betau1,
           gu2, betau2, gb1, betab1, gb2, betab2, gc, betac):
    xa, xh, bh = _run_p1(x, boundary_attr, W_mu, b_mu)
    t = _run_p2(up_attr, W_mu)
    ou, ob = _run_sc(xa, t, xh, bh, up_index, boundary_index)
    return _run_mlp(ou, ob, Wu1, bu1, Wu2, bu2, Wb1, bb1, Wb2, bb2, Wc, bc,
                    gu1, betau1, gu2, betau2, gb1, betab1, gb2, betab2,
                    gc, betac)
